# baseline (speedup 1.0000x reference)
import functools

import jax
import jax.numpy as jnp
from jax import lax
from jax.experimental import pallas as pl
from jax.experimental.pallas import tpu as pltpu
from jax.experimental.pallas import tpu_sc as plsc

NC = 2
NS = 16
NW = NC * NS
CHUNK = 128
NBUF = 10
LAG = 5


def _gather_kernel(n_chunks, n_per_w, D, src_hbm, table_hbm, out_hbm,
                   idx_v, bufs, gsems, ssems):
    wid = lax.axis_index("s") * NC + lax.axis_index("c")
    base = wid * n_per_w
    pltpu.sync_copy(src_hbm.at[wid], idx_v)

    def wait_gather(b):
        pltpu.make_async_copy(table_hbm.at[idx_v.at[0]], bufs[b],
                              gsems[b]).wait()

    def wait_store(b):
        pltpu.make_async_copy(bufs[b], out_hbm.at[pl.ds(base, CHUNK)],
                              ssems[b]).wait()

    @pl.loop(0, n_chunks, step=NBUF)
    def _(g):
        for i in range(NBUF):
            j = g + i
            b = i

            @pl.when(j >= NBUF)
            def _():
                wait_store(b)

            pltpu.async_copy(table_hbm.at[idx_v.at[j]], bufs[b], gsems[b])

            jj = j - LAG
            bb = (i - LAG) % NBUF

            @pl.when(jj >= 0)
            def _():
                wait_gather(bb)
                pltpu.async_copy(
                    bufs[bb], out_hbm.at[pl.ds(base + jj * CHUNK, CHUNK)],
                    ssems[bb])

    for i in range(LAG):
        jj = n_chunks - LAG + i
        bb = jj % NBUF
        wait_gather(bb)
        pltpu.async_copy(bufs[bb],
                         out_hbm.at[pl.ds(base + jj * CHUNK, CHUNK)],
                         ssems[bb])
    for b in range(NBUF):
        wait_store(b)


def kernel(src, table):
    B, S = src.shape
    V, D = table.shape
    N = B * S
    n_per_w = N // NW
    n_chunks = n_per_w // CHUNK
    idx = src.reshape(NW, n_chunks, CHUNK)

    mesh = plsc.VectorSubcoreMesh(core_axis_name="c", subcore_axis_name="s")
    run = functools.partial(
        pl.kernel,
        out_type=jax.ShapeDtypeStruct((N, D), jnp.float32),
        mesh=mesh,
        scratch_types=[
            pltpu.VMEM((n_chunks, CHUNK), jnp.int32),
            [pltpu.VMEM((CHUNK, D), jnp.float32) for _ in range(NBUF)],
            [pltpu.SemaphoreType.DMA for _ in range(NBUF)],
            [pltpu.SemaphoreType.DMA for _ in range(NBUF)],
        ],
        compiler_params=pltpu.CompilerParams(use_tc_tiling_on_sc=False),
    )(functools.partial(_gather_kernel, n_chunks, n_per_w, D))
    out = run(idx, table)
    return out.reshape(B, S, D)


# speedup vs baseline: 1.0007x; 1.0007x over previous
import functools

import jax
import jax.numpy as jnp
from jax import lax
from jax.experimental import pallas as pl
from jax.experimental.pallas import tpu as pltpu
from jax.experimental.pallas import tpu_sc as plsc

NC = 2
NS = 16
NW = NC * NS
C0 = 128
NRB = 4
LAG = 2


def _gather_kernel(rows_per_w, S, D, src_hbm, table_hbm, out_hbm,
                   idx_v, bufs, gsems, ssems):
    C1 = S - C0
    wid = lax.axis_index("s") * NC + lax.axis_index("c")
    base = wid * rows_per_w
    pltpu.sync_copy(src_hbm.at[pl.ds(base, rows_per_w)], idx_v)

    def start_gathers(r, b):
        pltpu.async_copy(table_hbm.at[idx_v.at[r, pl.ds(0, C0)]],
                         bufs[b].at[pl.ds(0, C0)], gsems[b])
        pltpu.async_copy(table_hbm.at[idx_v.at[r, pl.ds(C0, C1)]],
                         bufs[b].at[pl.ds(C0, C1)], gsems[b])

    def wait_gathers(b):
        pltpu.make_async_copy(table_hbm.at[idx_v.at[0, pl.ds(0, C0)]],
                              bufs[b].at[pl.ds(0, C0)], gsems[b]).wait()
        pltpu.make_async_copy(table_hbm.at[idx_v.at[0, pl.ds(C0, C1)]],
                              bufs[b].at[pl.ds(C0, C1)], gsems[b]).wait()

    def start_store(r, b):
        pltpu.async_copy(bufs[b], out_hbm.at[base + r], ssems[b])

    def wait_store(b):
        pltpu.make_async_copy(bufs[b], out_hbm.at[base], ssems[b]).wait()

    @pl.loop(0, rows_per_w, step=NRB)
    def _(g):
        for i in range(NRB):
            r = g + i
            b = i

            @pl.when(r >= NRB)
            def _():
                wait_store(b)

            start_gathers(r, b)

            rr = r - LAG
            bb = (i - LAG) % NRB

            @pl.when(rr >= 0)
            def _():
                wait_gathers(bb)
                start_store(rr, bb)

    for i in range(LAG):
        rr = rows_per_w - LAG + i
        bb = rr % NRB
        wait_gathers(bb)
        start_store(rr, bb)
    for b in range(NRB):
        wait_store(b)


def kernel(src, table):
    B, S = src.shape
    V, D = table.shape
    rows_per_w = B // NW

    mesh = plsc.VectorSubcoreMesh(core_axis_name="c", subcore_axis_name="s")
    run = functools.partial(
        pl.kernel,
        out_type=jax.ShapeDtypeStruct((B, S, D), jnp.float32),
        mesh=mesh,
        scratch_types=[
            pltpu.VMEM((rows_per_w, S), jnp.int32),
            [pltpu.VMEM((S, D), jnp.float32) for _ in range(NRB)],
            [pltpu.SemaphoreType.DMA for _ in range(NRB)],
            [pltpu.SemaphoreType.DMA for _ in range(NRB)],
        ],
        compiler_params=pltpu.CompilerParams(use_tc_tiling_on_sc=False),
    )(functools.partial(_gather_kernel, rows_per_w, S, D))
    return run(src, table)
